# baseline (speedup 1.0000x reference)
import functools

import jax
import jax.numpy as jnp
from jax import lax
from jax.experimental import pallas as pl
from jax.experimental.pallas import tpu as pltpu
from jax.experimental.pallas import tpu_sc as plsc

_ROWS, _F = 32768, 256
_NSPLITS = 17

_info = plsc.get_sparse_core_info()
_NC, _NS = _info.num_cores, _info.num_subcores
_NW = _NC * _NS
_RPW = _ROWS // _NW


@functools.partial(
    pl.kernel,
    mesh=plsc.VectorSubcoreMesh(core_axis_name="c", subcore_axis_name="s"),
    out_type=(
        jax.ShapeDtypeStruct((_ROWS, _F), jnp.float32),
        jax.ShapeDtypeStruct((_NSPLITS,), jnp.int32),
    ),
    scratch_types=[pltpu.SemaphoreType.DMA],
)
def _ragged_to_flat_sc(flat_hbm, cu_hbm, out_flat_hbm, out_cu_hbm, sem):
    wid = lax.axis_index("s") * _NC + lax.axis_index("c")
    base = wid * _RPW
    cp = pltpu.async_copy(
        flat_hbm.at[pl.ds(base, _RPW)],
        out_flat_hbm.at[pl.ds(base, _RPW)],
        sem,
    )

    @pl.when(wid == 0)
    def _():
        pltpu.sync_copy(cu_hbm, out_cu_hbm)

    cp.wait()


def kernel(flat, cu_seqlens):
    return _ragged_to_flat_sc(flat, cu_seqlens)


# speedup vs baseline: 1.0181x; 1.0181x over previous
import jax
import jax.numpy as jnp
from jax.experimental import pallas as pl
from jax.experimental.pallas import tpu as pltpu

_ROWS, _F = 32768, 256
_NSPLITS = 17


def _copy_body(flat_ref, cu_ref, out_flat_ref, out_cu_ref, sem_a, sem_b):
    c1 = pltpu.make_async_copy(flat_ref, out_flat_ref, sem_a)
    c2 = pltpu.make_async_copy(cu_ref, out_cu_ref, sem_b)
    c1.start()
    c2.start()
    c1.wait()
    c2.wait()


def kernel(flat, cu_seqlens):
    return pl.pallas_call(
        _copy_body,
        out_shape=(
            jax.ShapeDtypeStruct((_ROWS, _F), jnp.float32),
            jax.ShapeDtypeStruct((_NSPLITS,), jnp.int32),
        ),
        in_specs=[
            pl.BlockSpec(memory_space=pl.ANY),
            pl.BlockSpec(memory_space=pl.ANY),
        ],
        out_specs=(
            pl.BlockSpec(memory_space=pl.ANY),
            pl.BlockSpec(memory_space=pl.ANY),
        ),
        scratch_shapes=[pltpu.SemaphoreType.DMA, pltpu.SemaphoreType.DMA],
    )(flat, cu_seqlens)


# speedup vs baseline: 1.0185x; 1.0005x over previous
import jax
import jax.numpy as jnp
from jax.experimental import pallas as pl
from jax.experimental.pallas import tpu as pltpu

_ROWS, _F = 32768, 256
_NSPLITS = 17


_NCHUNKS = 32
_CR = _ROWS // _NCHUNKS


def _copy_body(flat_ref, cu_ref, out_flat_ref, out_cu_ref, sem_a, sem_b):
    copies = []
    for i in range(_NCHUNKS):
        c = pltpu.make_async_copy(
            flat_ref.at[pl.ds(i * _CR, _CR)],
            out_flat_ref.at[pl.ds(i * _CR, _CR)],
            sem_a,
        )
        c.start()
        copies.append(c)
    c2 = pltpu.make_async_copy(cu_ref, out_cu_ref, sem_b)
    c2.start()
    for c in copies:
        c.wait()
    c2.wait()


def kernel(flat, cu_seqlens):
    return pl.pallas_call(
        _copy_body,
        out_shape=(
            jax.ShapeDtypeStruct((_ROWS, _F), jnp.float32),
            jax.ShapeDtypeStruct((_NSPLITS,), jnp.int32),
        ),
        in_specs=[
            pl.BlockSpec(memory_space=pl.ANY),
            pl.BlockSpec(memory_space=pl.ANY),
        ],
        out_specs=(
            pl.BlockSpec(memory_space=pl.ANY),
            pl.BlockSpec(memory_space=pl.ANY),
        ),
        scratch_shapes=[pltpu.SemaphoreType.DMA, pltpu.SemaphoreType.DMA],
    )(flat, cu_seqlens)


# speedup vs baseline: 29.7398x; 29.1983x over previous
import jax
import jax.numpy as jnp
from jax.experimental import pallas as pl
from jax.experimental.pallas import tpu as pltpu

_ROWS, _F = 32768, 256
_NSPLITS = 17
_BR = 1024


def _values_body(flat_ref, out_ref):
    out_ref[...] = flat_ref[...]


def _splits_body(cu_ref, out_cu_ref, sem):
    c = pltpu.make_async_copy(cu_ref, out_cu_ref, sem)
    c.start()
    c.wait()


def kernel(flat, cu_seqlens):
    values = pl.pallas_call(
        _values_body,
        grid=(_ROWS // _BR,),
        in_specs=[pl.BlockSpec((_BR, _F), lambda i: (i, 0))],
        out_specs=pl.BlockSpec((_BR, _F), lambda i: (i, 0)),
        out_shape=jax.ShapeDtypeStruct((_ROWS, _F), jnp.float32),
    )(flat)
    row_splits = pl.pallas_call(
        _splits_body,
        out_shape=jax.ShapeDtypeStruct((_NSPLITS,), jnp.int32),
        in_specs=[pl.BlockSpec(memory_space=pl.ANY)],
        out_specs=pl.BlockSpec(memory_space=pl.ANY),
        scratch_shapes=[pltpu.SemaphoreType.DMA],
    )(cu_seqlens)
    return (values, row_splits)


# speedup vs baseline: 39.9618x; 1.3437x over previous
import jax
import jax.numpy as jnp
from jax.experimental import pallas as pl
from jax.experimental.pallas import tpu as pltpu

_ROWS, _F = 32768, 256
_NSPLITS = 17
_BR = 2048


def _copy_body(flat_ref, cu_ref, out_ref, out_cu_ref, sem):
    @pl.when(pl.program_id(0) == 0)
    def _():
        c = pltpu.make_async_copy(cu_ref, out_cu_ref, sem)
        c.start()
        c.wait()

    out_ref[...] = flat_ref[...]


def kernel(flat, cu_seqlens):
    return pl.pallas_call(
        _copy_body,
        grid=(_ROWS // _BR,),
        in_specs=[
            pl.BlockSpec((_BR, _F), lambda i: (i, 0)),
            pl.BlockSpec(memory_space=pl.ANY),
        ],
        out_specs=(
            pl.BlockSpec((_BR, _F), lambda i: (i, 0)),
            pl.BlockSpec(memory_space=pl.ANY),
        ),
        out_shape=(
            jax.ShapeDtypeStruct((_ROWS, _F), jnp.float32),
            jax.ShapeDtypeStruct((_NSPLITS,), jnp.int32),
        ),
        scratch_shapes=[pltpu.SemaphoreType.DMA],
    )(flat, cu_seqlens)


# speedup vs baseline: 43.5551x; 1.0899x over previous
import jax
import jax.numpy as jnp
from jax.experimental import pallas as pl
from jax.experimental.pallas import tpu as pltpu

_ROWS, _F = 32768, 256
_NSPLITS = 17
_BR = 4096


def _copy_body(flat_ref, cu_ref, out_ref, out_cu_ref, sem):
    @pl.when(pl.program_id(0) == 0)
    def _():
        c = pltpu.make_async_copy(cu_ref, out_cu_ref, sem)
        c.start()
        c.wait()

    out_ref[...] = flat_ref[...]


def kernel(flat, cu_seqlens):
    return pl.pallas_call(
        _copy_body,
        grid=(_ROWS // _BR,),
        in_specs=[
            pl.BlockSpec((_BR, _F), lambda i: (i, 0)),
            pl.BlockSpec(memory_space=pl.ANY),
        ],
        out_specs=(
            pl.BlockSpec((_BR, _F), lambda i: (i, 0)),
            pl.BlockSpec(memory_space=pl.ANY),
        ),
        out_shape=(
            jax.ShapeDtypeStruct((_ROWS, _F), jnp.float32),
            jax.ShapeDtypeStruct((_NSPLITS,), jnp.int32),
        ),
        scratch_shapes=[pltpu.SemaphoreType.DMA],
    )(flat, cu_seqlens)


# speedup vs baseline: 47.3754x; 1.0877x over previous
import jax
import jax.numpy as jnp
from jax.experimental import pallas as pl
from jax.experimental.pallas import tpu as pltpu

_ROWS, _F = 32768, 256
_NSPLITS = 17
_BR = 8192


def _copy_body(flat_ref, cu_ref, out_ref, out_cu_ref, sem):
    @pl.when(pl.program_id(0) == 0)
    def _():
        c = pltpu.make_async_copy(cu_ref, out_cu_ref, sem)
        c.start()
        c.wait()

    out_ref[...] = flat_ref[...]


def kernel(flat, cu_seqlens):
    return pl.pallas_call(
        _copy_body,
        grid=(_ROWS // _BR,),
        in_specs=[
            pl.BlockSpec((_BR, _F), lambda i: (i, 0)),
            pl.BlockSpec(memory_space=pl.ANY),
        ],
        out_specs=(
            pl.BlockSpec((_BR, _F), lambda i: (i, 0)),
            pl.BlockSpec(memory_space=pl.ANY),
        ),
        out_shape=(
            jax.ShapeDtypeStruct((_ROWS, _F), jnp.float32),
            jax.ShapeDtypeStruct((_NSPLITS,), jnp.int32),
        ),
        scratch_shapes=[pltpu.SemaphoreType.DMA],
    )(flat, cu_seqlens)
